# Initial kernel scaffold; baseline (speedup 1.0000x reference)
import jax
import jax.numpy as jnp
from jax.experimental import pallas as pl


def kernel(x, edge_index, batch, edge_attr, W1, b1, W2, b2, fc1_W, fc1_b, fc2_W, fc2_b):
    raise NotImplementedError("write your pallas kernel here")


# speedup vs baseline: 7.3672x; 7.3672x over previous
import functools

import jax
import jax.numpy as jnp
from jax import lax
from jax.experimental import pallas as pl
from jax.experimental.pallas import tpu as pltpu
from jax.experimental.pallas import tpu_sc as plsc

NC = 2
NS = 16
NW = NC * NS
LANES = 16
K = 128


def _mesh():
    return plsc.VectorSubcoreMesh(
        core_axis_name="c", subcore_axis_name="s",
        num_cores=NC, num_subcores=NS)


_SC_PARAMS = pltpu.CompilerParams(needs_layout_passes=False,
                                 use_tc_tiling_on_sc=False)


def _sc_deg(col, ea, n):
    E = col.shape[0]
    Ed = E // NS
    nbd = Ed // K
    rpt = n // NS

    @functools.partial(
        pl.kernel, mesh=_mesh(), compiler_params=_SC_PARAMS,
        out_type=jax.ShapeDtypeStruct((n,), jnp.float32),
        scratch_types=[
            pltpu.VMEM_SHARED((n,), jnp.float32),
            pltpu.VMEM((nbd, 1, K), jnp.int32),
            pltpu.VMEM((Ed,), jnp.float32),
        ],
    )
    def k(col3_h, ea_h, zeros_h, out_h, acc, cidx3, vals):
        cid = lax.axis_index("c")
        sid = lax.axis_index("s")
        pltpu.sync_copy(zeros_h.at[pl.ds(sid * rpt, rpt)],
                        acc.at[pl.ds(sid * rpt, rpt)])
        plsc.subcore_barrier()
        pltpu.sync_copy(col3_h.at[sid], cidx3)
        pltpu.sync_copy(ea_h.at[pl.ds(sid * Ed, Ed)], vals)

        def dbody(b, _):
            pltpu.sync_copy(vals.at[pl.ds(b * K, K)], acc.at[cidx3.at[b, 0]],
                            add=True)
            return 0
        lax.fori_loop(0, nbd, dbody, 0)
        plsc.subcore_barrier()

        @pl.when(cid == 0)
        def _():
            pltpu.sync_copy(acc.at[pl.ds(sid * rpt, rpt)],
                            out_h.at[pl.ds(sid * rpt, rpt)])

    col3 = col.reshape(NS, nbd, 1, K)
    return k(col3, ea, jnp.zeros((n,), jnp.float32))


def _rsqrt_body(d_ref, o_ref):
    o_ref[...] = lax.rsqrt(d_ref[...] + 1.0)


def _tc_rsqrt(deg, n):
    d2 = deg.reshape(128, n // 128)
    out = pl.pallas_call(
        _rsqrt_body,
        out_shape=jax.ShapeDtypeStruct(d2.shape, jnp.float32),
    )(d2)
    return out.reshape(n)


def _sc_norm(dis, row, col, ea, n):
    E = row.shape[0]
    Et = E // NW

    @functools.partial(
        pl.kernel, mesh=_mesh(), compiler_params=_SC_PARAMS,
        out_type=jax.ShapeDtypeStruct((E,), jnp.float32),
        scratch_types=[
            pltpu.VMEM((n,), jnp.float32),
            pltpu.VMEM((Et,), jnp.int32),
            pltpu.VMEM((Et,), jnp.int32),
            pltpu.VMEM((Et,), jnp.float32),
            pltpu.VMEM((Et,), jnp.float32),
        ],
    )
    def k(dis_h, row_h, col_h, ea_h, norm_h,
          p, ridx, cidx, eav, nbuf):
        cid = lax.axis_index("c")
        sid = lax.axis_index("s")
        w = cid * NS + sid
        pltpu.sync_copy(dis_h, p)

        base0 = w * Et
        pltpu.sync_copy(row_h.at[pl.ds(base0, Et)], ridx)
        pltpu.sync_copy(col_h.at[pl.ds(base0, Et)], cidx)
        pltpu.sync_copy(ea_h.at[pl.ds(base0, Et)], eav)

        def g(gi, _):
            sl = pl.ds(gi * LANES, LANES)
            dr = plsc.load_gather(p, [ridx[sl]])
            dc = plsc.load_gather(p, [cidx[sl]])
            nbuf[sl] = dr * eav[sl] * dc
            return 0
        lax.fori_loop(0, Et // LANES, g, 0, unroll=4)
        pltpu.sync_copy(nbuf, norm_h.at[pl.ds(base0, Et)])

    return k(dis, row, col, ea)


def _sc_spmm(h_chunks, row, col4, norm, zeros, n):
    C = len(h_chunks)
    E = row.shape[0]
    Dc = h_chunks[0].shape[1]
    NV = Dc // LANES
    Et = E // NW
    nb = Et // K
    rpt = n // NS

    @functools.partial(
        pl.kernel, mesh=_mesh(), compiler_params=_SC_PARAMS,
        out_type=jax.ShapeDtypeStruct((C, NC, n, Dc), jnp.float32),
        scratch_types=(
            [pltpu.VMEM_SHARED((n, Dc), jnp.float32),
             pltpu.VMEM((Et,), jnp.int32),
             pltpu.VMEM((nb, 1, K), jnp.int32),
             pltpu.VMEM((Et,), jnp.float32)]
            + [pltpu.VMEM((K, Dc), jnp.float32)] * 2
            + [pltpu.SemaphoreType.DMA] * 2
        ),
    )
    def k(*refs):
        h_refs = refs[:C]
        row_h, col_h, norm_h, zeros_h, out_h = refs[C:C + 5]
        acc, ridx, cidx, nrm, rows0, rows1, sem0, sem1 = refs[C + 5:]
        rows = (rows0, rows1)
        sems = (sem0, sem1)
        cid = lax.axis_index("c")
        sid = lax.axis_index("s")
        w = cid * NS + sid
        estart = w * Et

        pltpu.sync_copy(row_h.at[pl.ds(estart, Et)], ridx)
        pltpu.sync_copy(col_h.at[w], cidx)
        pltpu.sync_copy(norm_h.at[pl.ds(estart, Et)], nrm)

        def scale_batch(b, buf):
            def scale16(g, _):
                n16 = nrm[pl.ds(b * K + g * LANES, LANES)]
                for u in range(LANES):
                    s = n16[u]
                    e = g * LANES + u
                    for j in range(NV):
                        sl = pl.ds(j * LANES, LANES)
                        buf[e, sl] = buf[e, sl] * s
                return 0
            lax.fori_loop(0, K // LANES, scale16, 0)

        for c in range(C):
            pltpu.sync_copy(zeros_h.at[pl.ds(sid * rpt, rpt)],
                            acc.at[pl.ds(sid * rpt, rpt)])
            plsc.subcore_barrier()
            pltpu.async_copy(h_refs[c].at[ridx.at[pl.ds(0, K)]],
                             rows0, sem0)

            def body(b, _):
                slot = lax.rem(b, 2)

                @pl.when(b + 1 < nb)
                def _():
                    for s_ in range(2):
                        @pl.when(slot != s_)
                        def _():
                            pltpu.async_copy(
                                h_refs[c].at[ridx.at[pl.ds((b + 1) * K, K)]],
                                rows[s_], sems[s_])
                for s_ in range(2):
                    @pl.when(slot == s_)
                    def _():
                        pltpu.make_async_copy(
                            h_refs[c].at[ridx.at[pl.ds(0, K)]],
                            rows[s_], sems[s_]).wait()
                        scale_batch(b, rows[s_])
                        pltpu.sync_copy(rows[s_], acc.at[cidx.at[b, 0]],
                                        add=True)
                return 0
            lax.fori_loop(0, nb, body, 0)
            plsc.subcore_barrier()
            pltpu.sync_copy(acc.at[pl.ds(sid * rpt, rpt)],
                            out_h.at[c, cid, pl.ds(sid * rpt, rpt)])

    return k(*h_chunks, row, col4, norm, zeros)


def _mm_body(C, Dc, x_ref, w_ref, o_ref):
    r = jnp.dot(x_ref[...], w_ref[...], preferred_element_type=jnp.float32)
    for cc in range(C):
        o_ref[cc] = r[:, cc * Dc:(cc + 1) * Dc]


def _tc_matmul_chunked(x, W, C):
    N, KD = x.shape
    D = W.shape[1]
    Dc = D // C
    BN = 512
    return pl.pallas_call(
        functools.partial(_mm_body, C, Dc),
        grid=(N // BN,),
        in_specs=[pl.BlockSpec((BN, KD), lambda i: (i, 0)),
                  pl.BlockSpec((KD, D), lambda i: (0, 0))],
        out_specs=pl.BlockSpec((C, BN, Dc), lambda i: (0, i, 0)),
        out_shape=jax.ShapeDtypeStruct((C, N, Dc), jnp.float32),
    )(x, W)


def _combine_mm_body(Cin, Cout, Dc, agg_ref, hc_ref, dis_ref, b_ref, w_ref,
                     o_ref):
    dinv = dis_ref[...] * dis_ref[...]
    acc = None
    for c in range(Cin):
        part = (agg_ref[c, 0] + agg_ref[c, 1] + hc_ref[c] * dinv
                + b_ref[0, pl.ds(c * Dc, Dc)][None, :])
        part = jnp.maximum(part, 0.0)
        pc = jnp.dot(part, w_ref[pl.ds(c * Dc, Dc), :],
                     preferred_element_type=jnp.float32)
        acc = pc if acc is None else acc + pc
    Dco = acc.shape[1] // Cout
    for cc in range(Cout):
        o_ref[cc] = acc[:, cc * Dco:(cc + 1) * Dco]


def _tc_combine_mm(agg, hc, dis, b, W, Cout):
    Cin, _, N, Dc = agg.shape
    D = W.shape[1]
    Dco = D // Cout
    BN = 512
    return pl.pallas_call(
        functools.partial(_combine_mm_body, Cin, Cout, Dc),
        grid=(N // BN,),
        in_specs=[pl.BlockSpec((Cin, NC, BN, Dc), lambda i: (0, 0, i, 0)),
                  pl.BlockSpec((Cin, BN, Dc), lambda i: (0, i, 0)),
                  pl.BlockSpec((BN, 1), lambda i: (i, 0)),
                  pl.BlockSpec((1, Cin * Dc), lambda i: (0, 0)),
                  pl.BlockSpec((Cin * Dc, D), lambda i: (0, 0))],
        out_specs=pl.BlockSpec((Cout, BN, Dco), lambda i: (0, i, 0)),
        out_shape=jax.ShapeDtypeStruct((Cout, N, Dco), jnp.float32),
    )(agg, hc, dis, b, W)


def _combine_body(Cin, Dc, agg_ref, hc_ref, dis_ref, b_ref, o_ref):
    dinv = dis_ref[...] * dis_ref[...]
    for c in range(Cin):
        part = (agg_ref[c, 0] + agg_ref[c, 1] + hc_ref[c] * dinv
                + b_ref[0, pl.ds(c * Dc, Dc)][None, :])
        o_ref[:, pl.ds(c * Dc, Dc)] = jnp.maximum(part, 0.0)


def _tc_combine(agg, hc, dis, b):
    Cin, _, N, Dc = agg.shape
    BN = 512
    return pl.pallas_call(
        functools.partial(_combine_body, Cin, Dc),
        grid=(N // BN,),
        in_specs=[pl.BlockSpec((Cin, NC, BN, Dc), lambda i: (0, 0, i, 0)),
                  pl.BlockSpec((Cin, BN, Dc), lambda i: (0, i, 0)),
                  pl.BlockSpec((BN, 1), lambda i: (i, 0)),
                  pl.BlockSpec((1, Cin * Dc), lambda i: (0, 0))],
        out_specs=pl.BlockSpec((BN, Cin * Dc), lambda i: (i, 0)),
        out_shape=jax.ShapeDtypeStruct((N, Cin * Dc), jnp.float32),
    )(agg, hc, dis, b)


def _head_body(l_ref, w1_ref, b1_ref, w2_ref, b2_ref, o_ref, acc_ref):
    kk = pl.program_id(0)

    @pl.when(kk == 0)
    def _():
        acc_ref[...] = jnp.zeros_like(acc_ref)

    acc_ref[...] += jnp.dot(l_ref[...], w1_ref[...],
                            preferred_element_type=jnp.float32)

    @pl.when(kk == pl.num_programs(0) - 1)
    def _():
        r = jnp.maximum(acc_ref[...] + b1_ref[...], 0.0)
        o_ref[...] = jnp.dot(r, w2_ref[...],
                             preferred_element_type=jnp.float32) + b2_ref[...]


def _tc_head(L, fc1_W, fc1_b, fc2_W, fc2_b):
    Bsz, KD = L.shape
    H = fc1_W.shape[1]
    O = fc2_W.shape[1]
    BK = 2048
    return pl.pallas_call(
        _head_body,
        grid=(KD // BK,),
        in_specs=[pl.BlockSpec((Bsz, BK), lambda k_: (0, k_)),
                  pl.BlockSpec((BK, H), lambda k_: (k_, 0)),
                  pl.BlockSpec((1, H), lambda k_: (0, 0)),
                  pl.BlockSpec((H, O), lambda k_: (0, 0)),
                  pl.BlockSpec((1, O), lambda k_: (0, 0))],
        out_specs=pl.BlockSpec((Bsz, O), lambda k_: (0, 0)),
        out_shape=jax.ShapeDtypeStruct((Bsz, O), jnp.float32),
        scratch_shapes=[pltpu.VMEM((Bsz, H), jnp.float32)],
    )(L, fc1_W, fc1_b, fc2_W, fc2_b)


def kernel(x, edge_index, batch, edge_attr, W1, b1, W2, b2,
           fc1_W, fc1_b, fc2_W, fc2_b):
    N = x.shape[0]
    E = edge_index.shape[1]
    row = edge_index[0]
    col = edge_index[1]

    deg = _sc_deg(col, edge_attr, N)
    dis = _tc_rsqrt(deg, N)
    norm = _sc_norm(dis, row, col, edge_attr, N)
    dis2 = dis.reshape(N, 1)
    col4 = col.reshape(NW, (E // NW) // K, 1, K)
    zeros = jnp.zeros((N, 64), jnp.float32)

    C1 = W1.shape[1] // 64
    h1c = _tc_matmul_chunked(x, W1, C1)
    agg1 = _sc_spmm([h1c[i] for i in range(C1)], row, col4, norm, zeros, N)

    C2 = W2.shape[1] // 64
    h2c = _tc_combine_mm(agg1, h1c, dis2, b1.reshape(1, -1), W2, C2)
    agg2 = _sc_spmm([h2c[i] for i in range(C2)], row, col4, norm, zeros, N)

    hf = _tc_combine(agg2, h2c, dis2, b2.reshape(1, -1))

    npg = fc1_W.shape[0] // hf.shape[1]
    Bsz = N // npg
    L = hf.reshape(Bsz, -1)
    return _tc_head(L, fc1_W, fc1_b.reshape(1, -1),
                    fc2_W, fc2_b.reshape(1, -1))


# speedup vs baseline: 7.3815x; 1.0019x over previous
import functools

import jax
import jax.numpy as jnp
from jax import lax
from jax.experimental import pallas as pl
from jax.experimental.pallas import tpu as pltpu
from jax.experimental.pallas import tpu_sc as plsc

NC = 2
NS = 16
NW = NC * NS
LANES = 16
K = 128


def _mesh():
    return plsc.VectorSubcoreMesh(
        core_axis_name="c", subcore_axis_name="s",
        num_cores=NC, num_subcores=NS)


_SC_PARAMS = pltpu.CompilerParams(needs_layout_passes=False,
                                 use_tc_tiling_on_sc=False)


def _sc_deg(col, ea, n):
    E = col.shape[0]
    Ed = E // NS
    nbd = Ed // K
    rpt = n // NS

    @functools.partial(
        pl.kernel, mesh=_mesh(), compiler_params=_SC_PARAMS,
        out_type=jax.ShapeDtypeStruct((n,), jnp.float32),
        scratch_types=[
            pltpu.VMEM_SHARED((n,), jnp.float32),
            pltpu.VMEM((nbd, 1, K), jnp.int32),
            pltpu.VMEM((Ed,), jnp.float32),
        ],
    )
    def k(col3_h, ea_h, zeros_h, out_h, acc, cidx3, vals):
        cid = lax.axis_index("c")
        sid = lax.axis_index("s")
        pltpu.sync_copy(zeros_h.at[pl.ds(sid * rpt, rpt)],
                        acc.at[pl.ds(sid * rpt, rpt)])
        plsc.subcore_barrier()
        pltpu.sync_copy(col3_h.at[sid], cidx3)
        pltpu.sync_copy(ea_h.at[pl.ds(sid * Ed, Ed)], vals)

        def dbody(b, _):
            pltpu.sync_copy(vals.at[pl.ds(b * K, K)], acc.at[cidx3.at[b, 0]],
                            add=True)
            return 0
        lax.fori_loop(0, nbd, dbody, 0)
        plsc.subcore_barrier()

        @pl.when(cid == 0)
        def _():
            pltpu.sync_copy(acc.at[pl.ds(sid * rpt, rpt)],
                            out_h.at[pl.ds(sid * rpt, rpt)])

    col3 = col.reshape(NS, nbd, 1, K)
    return k(col3, ea, jnp.zeros((n,), jnp.float32))


def _rsqrt_body(d_ref, o_ref):
    o_ref[...] = lax.rsqrt(d_ref[...] + 1.0)


def _tc_rsqrt(deg, n):
    d2 = deg.reshape(128, n // 128)
    out = pl.pallas_call(
        _rsqrt_body,
        out_shape=jax.ShapeDtypeStruct(d2.shape, jnp.float32),
    )(d2)
    return out.reshape(n)


def _sc_norm(dis, row, col, ea, n):
    E = row.shape[0]
    Et = E // NW

    @functools.partial(
        pl.kernel, mesh=_mesh(), compiler_params=_SC_PARAMS,
        out_type=jax.ShapeDtypeStruct((E,), jnp.float32),
        scratch_types=[
            pltpu.VMEM((n,), jnp.float32),
            pltpu.VMEM((Et,), jnp.int32),
            pltpu.VMEM((Et,), jnp.int32),
            pltpu.VMEM((Et,), jnp.float32),
            pltpu.VMEM((Et,), jnp.float32),
        ],
    )
    def k(dis_h, row_h, col_h, ea_h, norm_h,
          p, ridx, cidx, eav, nbuf):
        cid = lax.axis_index("c")
        sid = lax.axis_index("s")
        w = cid * NS + sid
        pltpu.sync_copy(dis_h, p)

        base0 = w * Et
        pltpu.sync_copy(row_h.at[pl.ds(base0, Et)], ridx)
        pltpu.sync_copy(col_h.at[pl.ds(base0, Et)], cidx)
        pltpu.sync_copy(ea_h.at[pl.ds(base0, Et)], eav)

        def g(gi, _):
            sl = pl.ds(gi * LANES, LANES)
            dr = plsc.load_gather(p, [ridx[sl]])
            dc = plsc.load_gather(p, [cidx[sl]])
            nbuf[sl] = dr * eav[sl] * dc
            return 0
        lax.fori_loop(0, Et // LANES, g, 0, unroll=4)
        pltpu.sync_copy(nbuf, norm_h.at[pl.ds(base0, Et)])

    return k(dis, row, col, ea)


def _sc_spmm(h_chunks, row, col4, norm, zeros, n):
    C = len(h_chunks)
    E = row.shape[0]
    Dc = h_chunks[0].shape[1]
    NV = Dc // LANES
    Et = E // NW
    nb = Et // K
    rpt = n // NS

    @functools.partial(
        pl.kernel, mesh=_mesh(), compiler_params=_SC_PARAMS,
        out_type=jax.ShapeDtypeStruct((C, NC, n, Dc), jnp.float32),
        scratch_types=(
            [pltpu.VMEM_SHARED((n, Dc), jnp.float32),
             pltpu.VMEM((Et,), jnp.int32),
             pltpu.VMEM((nb, 1, K), jnp.int32),
             pltpu.VMEM((Et,), jnp.float32)]
            + [pltpu.VMEM((K, Dc), jnp.float32)] * 2
            + [pltpu.SemaphoreType.DMA] * 4
        ),
    )
    def k(*refs):
        h_refs = refs[:C]
        row_h, col_h, norm_h, zeros_h, out_h = refs[C:C + 5]
        (acc, ridx, cidx, nrm, rows0, rows1,
         sem0, sem1, ssem0, ssem1) = refs[C + 5:]
        rows = (rows0, rows1)
        sems = (sem0, sem1)
        ssems = (ssem0, ssem1)
        cid = lax.axis_index("c")
        sid = lax.axis_index("s")
        w = cid * NS + sid
        estart = w * Et

        pltpu.sync_copy(row_h.at[pl.ds(estart, Et)], ridx)
        pltpu.sync_copy(col_h.at[w], cidx)
        pltpu.sync_copy(norm_h.at[pl.ds(estart, Et)], nrm)

        def scale_batch(b, buf):
            def scale16(g, _):
                n16 = nrm[pl.ds(b * K + g * LANES, LANES)]
                for u in range(LANES):
                    s = n16[u]
                    e = g * LANES + u
                    for j in range(NV):
                        sl = pl.ds(j * LANES, LANES)
                        buf[e, sl] = buf[e, sl] * s
                return 0
            lax.fori_loop(0, K // LANES, scale16, 0)

        for c in range(C):
            pltpu.sync_copy(zeros_h.at[pl.ds(sid * rpt, rpt)],
                            acc.at[pl.ds(sid * rpt, rpt)])
            plsc.subcore_barrier()
            pltpu.async_copy(h_refs[c].at[ridx.at[pl.ds(0, K)]],
                             rows0, sem0)

            def body(b, _):
                slot = lax.rem(b, 2)
                for s_ in range(2):
                    o_ = 1 - s_

                    @pl.when(slot == s_)
                    def _():
                        @pl.when(b + 1 < nb)
                        def _():
                            @pl.when(b >= 1)
                            def _():
                                pltpu.make_async_copy(
                                    rows[o_], acc.at[cidx.at[0, 0]],
                                    ssems[o_]).wait()
                            pltpu.async_copy(
                                h_refs[c].at[ridx.at[pl.ds((b + 1) * K, K)]],
                                rows[o_], sems[o_])
                        pltpu.make_async_copy(
                            h_refs[c].at[ridx.at[pl.ds(0, K)]],
                            rows[s_], sems[s_]).wait()
                        scale_batch(b, rows[s_])
                        pltpu.async_copy(rows[s_], acc.at[cidx.at[b, 0]],
                                         ssems[s_], add=True)
                return 0
            lax.fori_loop(0, nb, body, 0)
            for s_ in range(2):
                pltpu.make_async_copy(rows[s_], acc.at[cidx.at[0, 0]],
                                      ssems[s_]).wait()
            plsc.subcore_barrier()
            pltpu.sync_copy(acc.at[pl.ds(sid * rpt, rpt)],
                            out_h.at[c, cid, pl.ds(sid * rpt, rpt)])

    return k(*h_chunks, row, col4, norm, zeros)


def _mm_body(C, Dc, x_ref, w_ref, o_ref):
    r = jnp.dot(x_ref[...], w_ref[...], preferred_element_type=jnp.float32)
    for cc in range(C):
        o_ref[cc] = r[:, cc * Dc:(cc + 1) * Dc]


def _tc_matmul_chunked(x, W, C):
    N, KD = x.shape
    D = W.shape[1]
    Dc = D // C
    BN = 512
    return pl.pallas_call(
        functools.partial(_mm_body, C, Dc),
        grid=(N // BN,),
        in_specs=[pl.BlockSpec((BN, KD), lambda i: (i, 0)),
                  pl.BlockSpec((KD, D), lambda i: (0, 0))],
        out_specs=pl.BlockSpec((C, BN, Dc), lambda i: (0, i, 0)),
        out_shape=jax.ShapeDtypeStruct((C, N, Dc), jnp.float32),
    )(x, W)


def _combine_mm_body(Cin, Cout, Dc, agg_ref, hc_ref, dis_ref, b_ref, w_ref,
                     o_ref):
    dinv = dis_ref[...] * dis_ref[...]
    acc = None
    for c in range(Cin):
        part = (agg_ref[c, 0] + agg_ref[c, 1] + hc_ref[c] * dinv
                + b_ref[0, pl.ds(c * Dc, Dc)][None, :])
        part = jnp.maximum(part, 0.0)
        pc = jnp.dot(part, w_ref[pl.ds(c * Dc, Dc), :],
                     preferred_element_type=jnp.float32)
        acc = pc if acc is None else acc + pc
    Dco = acc.shape[1] // Cout
    for cc in range(Cout):
        o_ref[cc] = acc[:, cc * Dco:(cc + 1) * Dco]


def _tc_combine_mm(agg, hc, dis, b, W, Cout):
    Cin, _, N, Dc = agg.shape
    D = W.shape[1]
    Dco = D // Cout
    BN = 512
    return pl.pallas_call(
        functools.partial(_combine_mm_body, Cin, Cout, Dc),
        grid=(N // BN,),
        in_specs=[pl.BlockSpec((Cin, NC, BN, Dc), lambda i: (0, 0, i, 0)),
                  pl.BlockSpec((Cin, BN, Dc), lambda i: (0, i, 0)),
                  pl.BlockSpec((BN, 1), lambda i: (i, 0)),
                  pl.BlockSpec((1, Cin * Dc), lambda i: (0, 0)),
                  pl.BlockSpec((Cin * Dc, D), lambda i: (0, 0))],
        out_specs=pl.BlockSpec((Cout, BN, Dco), lambda i: (0, i, 0)),
        out_shape=jax.ShapeDtypeStruct((Cout, N, Dco), jnp.float32),
    )(agg, hc, dis, b, W)


def _combine_body(Cin, Dc, agg_ref, hc_ref, dis_ref, b_ref, o_ref):
    dinv = dis_ref[...] * dis_ref[...]
    for c in range(Cin):
        part = (agg_ref[c, 0] + agg_ref[c, 1] + hc_ref[c] * dinv
                + b_ref[0, pl.ds(c * Dc, Dc)][None, :])
        o_ref[:, pl.ds(c * Dc, Dc)] = jnp.maximum(part, 0.0)


def _tc_combine(agg, hc, dis, b):
    Cin, _, N, Dc = agg.shape
    BN = 512
    return pl.pallas_call(
        functools.partial(_combine_body, Cin, Dc),
        grid=(N // BN,),
        in_specs=[pl.BlockSpec((Cin, NC, BN, Dc), lambda i: (0, 0, i, 0)),
                  pl.BlockSpec((Cin, BN, Dc), lambda i: (0, i, 0)),
                  pl.BlockSpec((BN, 1), lambda i: (i, 0)),
                  pl.BlockSpec((1, Cin * Dc), lambda i: (0, 0))],
        out_specs=pl.BlockSpec((BN, Cin * Dc), lambda i: (i, 0)),
        out_shape=jax.ShapeDtypeStruct((N, Cin * Dc), jnp.float32),
    )(agg, hc, dis, b)


def _head_body(l_ref, w1_ref, b1_ref, w2_ref, b2_ref, o_ref, acc_ref):
    kk = pl.program_id(0)

    @pl.when(kk == 0)
    def _():
        acc_ref[...] = jnp.zeros_like(acc_ref)

    acc_ref[...] += jnp.dot(l_ref[...], w1_ref[...],
                            preferred_element_type=jnp.float32)

    @pl.when(kk == pl.num_programs(0) - 1)
    def _():
        r = jnp.maximum(acc_ref[...] + b1_ref[...], 0.0)
        o_ref[...] = jnp.dot(r, w2_ref[...],
                             preferred_element_type=jnp.float32) + b2_ref[...]


def _tc_head(L, fc1_W, fc1_b, fc2_W, fc2_b):
    Bsz, KD = L.shape
    H = fc1_W.shape[1]
    O = fc2_W.shape[1]
    BK = 2048
    return pl.pallas_call(
        _head_body,
        grid=(KD // BK,),
        in_specs=[pl.BlockSpec((Bsz, BK), lambda k_: (0, k_)),
                  pl.BlockSpec((BK, H), lambda k_: (k_, 0)),
                  pl.BlockSpec((1, H), lambda k_: (0, 0)),
                  pl.BlockSpec((H, O), lambda k_: (0, 0)),
                  pl.BlockSpec((1, O), lambda k_: (0, 0))],
        out_specs=pl.BlockSpec((Bsz, O), lambda k_: (0, 0)),
        out_shape=jax.ShapeDtypeStruct((Bsz, O), jnp.float32),
        scratch_shapes=[pltpu.VMEM((Bsz, H), jnp.float32)],
    )(L, fc1_W, fc1_b, fc2_W, fc2_b)


def kernel(x, edge_index, batch, edge_attr, W1, b1, W2, b2,
           fc1_W, fc1_b, fc2_W, fc2_b):
    N = x.shape[0]
    E = edge_index.shape[1]
    row = edge_index[0]
    col = edge_index[1]

    deg = _sc_deg(col, edge_attr, N)
    dis = _tc_rsqrt(deg, N)
    norm = _sc_norm(dis, row, col, edge_attr, N)
    dis2 = dis.reshape(N, 1)
    col4 = col.reshape(NW, (E // NW) // K, 1, K)
    zeros = jnp.zeros((N, 64), jnp.float32)

    C1 = W1.shape[1] // 64
    h1c = _tc_matmul_chunked(x, W1, C1)
    agg1 = _sc_spmm([h1c[i] for i in range(C1)], row, col4, norm, zeros, N)

    C2 = W2.shape[1] // 64
    h2c = _tc_combine_mm(agg1, h1c, dis2, b1.reshape(1, -1), W2, C2)
    agg2 = _sc_spmm([h2c[i] for i in range(C2)], row, col4, norm, zeros, N)

    hf = _tc_combine(agg2, h2c, dis2, b2.reshape(1, -1))

    npg = fc1_W.shape[0] // hf.shape[1]
    Bsz = N // npg
    L = hf.reshape(Bsz, -1)
    return _tc_head(L, fc1_W, fc1_b.reshape(1, -1),
                    fc2_W, fc2_b.reshape(1, -1))


# speedup vs baseline: 13.4723x; 1.8251x over previous
import functools

import jax
import jax.numpy as jnp
from jax import lax
from jax.experimental import pallas as pl
from jax.experimental.pallas import tpu as pltpu
from jax.experimental.pallas import tpu_sc as plsc

NC = 2
NS = 16
NW = NC * NS
LANES = 16
K = 128


def _mesh():
    return plsc.VectorSubcoreMesh(
        core_axis_name="c", subcore_axis_name="s",
        num_cores=NC, num_subcores=NS)


_SC_PARAMS = pltpu.CompilerParams(needs_layout_passes=False,
                                 use_tc_tiling_on_sc=False)


def _sc_deg(col, ea, n):
    E = col.shape[0]
    Ed = E // NS
    nbd = Ed // K
    rpt = n // NS

    @functools.partial(
        pl.kernel, mesh=_mesh(), compiler_params=_SC_PARAMS,
        out_type=jax.ShapeDtypeStruct((n,), jnp.float32),
        scratch_types=[
            pltpu.VMEM_SHARED((n,), jnp.float32),
            pltpu.VMEM((nbd, 1, K), jnp.int32),
            pltpu.VMEM((Ed,), jnp.float32),
        ],
    )
    def k(col3_h, ea_h, zeros_h, out_h, acc, cidx3, vals):
        cid = lax.axis_index("c")
        sid = lax.axis_index("s")
        pltpu.sync_copy(zeros_h.at[pl.ds(sid * rpt, rpt)],
                        acc.at[pl.ds(sid * rpt, rpt)])
        plsc.subcore_barrier()
        pltpu.sync_copy(col3_h.at[sid], cidx3)
        pltpu.sync_copy(ea_h.at[pl.ds(sid * Ed, Ed)], vals)

        def dbody(b, _):
            pltpu.sync_copy(vals.at[pl.ds(b * K, K)], acc.at[cidx3.at[b, 0]],
                            add=True)
            return 0
        lax.fori_loop(0, nbd, dbody, 0)
        plsc.subcore_barrier()

        @pl.when(cid == 0)
        def _():
            pltpu.sync_copy(acc.at[pl.ds(sid * rpt, rpt)],
                            out_h.at[pl.ds(sid * rpt, rpt)])

    col3 = col.reshape(NS, nbd, 1, K)
    return k(col3, ea, jnp.zeros((n,), jnp.float32))


def _rsqrt_body(d_ref, o_ref):
    o_ref[...] = lax.rsqrt(d_ref[...] + 1.0)


def _tc_rsqrt(deg, n):
    d2 = deg.reshape(128, n // 128)
    out = pl.pallas_call(
        _rsqrt_body,
        out_shape=jax.ShapeDtypeStruct(d2.shape, jnp.float32),
    )(d2)
    return out.reshape(n)


def _sc_norm(dis, row, col, ea, n):
    E = row.shape[0]
    Et = E // NW

    @functools.partial(
        pl.kernel, mesh=_mesh(), compiler_params=_SC_PARAMS,
        out_type=jax.ShapeDtypeStruct((E,), jnp.float32),
        scratch_types=[
            pltpu.VMEM((n,), jnp.float32),
            pltpu.VMEM((Et,), jnp.int32),
            pltpu.VMEM((Et,), jnp.int32),
            pltpu.VMEM((Et,), jnp.float32),
            pltpu.VMEM((Et,), jnp.float32),
        ],
    )
    def k(dis_h, row_h, col_h, ea_h, norm_h,
          p, ridx, cidx, eav, nbuf):
        cid = lax.axis_index("c")
        sid = lax.axis_index("s")
        w = cid * NS + sid
        pltpu.sync_copy(dis_h, p)

        base0 = w * Et
        pltpu.sync_copy(row_h.at[pl.ds(base0, Et)], ridx)
        pltpu.sync_copy(col_h.at[pl.ds(base0, Et)], cidx)
        pltpu.sync_copy(ea_h.at[pl.ds(base0, Et)], eav)

        def g(gi, _):
            sl = pl.ds(gi * LANES, LANES)
            dr = plsc.load_gather(p, [ridx[sl]])
            dc = plsc.load_gather(p, [cidx[sl]])
            nbuf[sl] = dr * eav[sl] * dc
            return 0
        lax.fori_loop(0, Et // LANES, g, 0, unroll=4)
        pltpu.sync_copy(nbuf, norm_h.at[pl.ds(base0, Et)])

    return k(dis, row, col, ea)


def _sc_spmm(h_chunks, row, col4, norm, zeros, n):
    C = len(h_chunks)
    E = row.shape[0]
    Dc = h_chunks[0].shape[1]
    NV = Dc // LANES
    Et = E // NW
    nb = Et // K
    rpt = n // NS

    @functools.partial(
        pl.kernel, mesh=_mesh(), compiler_params=_SC_PARAMS,
        out_type=jax.ShapeDtypeStruct((C, NC, n, Dc), jnp.float32),
        scratch_types=(
            [pltpu.VMEM_SHARED((n, Dc), jnp.float32),
             pltpu.VMEM((Et,), jnp.int32),
             pltpu.VMEM((nb, 1, K), jnp.int32),
             pltpu.VMEM((Et,), jnp.float32)]
            + [pltpu.VMEM((K, Dc), jnp.float32)] * 2
            + [pltpu.SemaphoreType.DMA] * 4
        ),
    )
    def k(*refs):
        h_refs = refs[:C]
        row_h, col_h, norm_h, zeros_h, out_h = refs[C:C + 5]
        (acc, ridx, cidx, nrm, rows0, rows1,
         sem0, sem1, ssem0, ssem1) = refs[C + 5:]
        rows = (rows0, rows1)
        sems = (sem0, sem1)
        ssems = (ssem0, ssem1)
        cid = lax.axis_index("c")
        sid = lax.axis_index("s")
        w = cid * NS + sid
        estart = w * Et

        pltpu.sync_copy(row_h.at[pl.ds(estart, Et)], ridx)
        pltpu.sync_copy(col_h.at[w], cidx)
        pltpu.sync_copy(norm_h.at[pl.ds(estart, Et)], nrm)

        def scale_batch(b, buf):
            @plsc.parallel_loop(0, K // LANES, 1, unroll=2)
            def _(g):
                n16 = nrm[pl.ds(b * K + g * LANES, LANES)]
                for u in range(LANES):
                    s = n16[u]
                    e = g * LANES + u
                    for j in range(NV):
                        sl = pl.ds(j * LANES, LANES)
                        buf[e, sl] = buf[e, sl] * s

        for c in range(C):
            pltpu.sync_copy(zeros_h.at[pl.ds(sid * rpt, rpt)],
                            acc.at[pl.ds(sid * rpt, rpt)])
            plsc.subcore_barrier()
            pltpu.async_copy(h_refs[c].at[ridx.at[pl.ds(0, K)]],
                             rows0, sem0)

            def body(b, _):
                slot = lax.rem(b, 2)
                for s_ in range(2):
                    o_ = 1 - s_

                    @pl.when(slot == s_)
                    def _():
                        @pl.when(b + 1 < nb)
                        def _():
                            @pl.when(b >= 1)
                            def _():
                                pltpu.make_async_copy(
                                    rows[o_], acc.at[cidx.at[0, 0]],
                                    ssems[o_]).wait()
                            pltpu.async_copy(
                                h_refs[c].at[ridx.at[pl.ds((b + 1) * K, K)]],
                                rows[o_], sems[o_])
                        pltpu.make_async_copy(
                            h_refs[c].at[ridx.at[pl.ds(0, K)]],
                            rows[s_], sems[s_]).wait()
                        scale_batch(b, rows[s_])
                        pltpu.async_copy(rows[s_], acc.at[cidx.at[b, 0]],
                                         ssems[s_], add=True)
                return 0
            lax.fori_loop(0, nb, body, 0)
            for s_ in range(2):
                pltpu.make_async_copy(rows[s_], acc.at[cidx.at[0, 0]],
                                      ssems[s_]).wait()
            plsc.subcore_barrier()
            pltpu.sync_copy(acc.at[pl.ds(sid * rpt, rpt)],
                            out_h.at[c, cid, pl.ds(sid * rpt, rpt)])

    return k(*h_chunks, row, col4, norm, zeros)


def _mm_body(C, Dc, x_ref, w_ref, o_ref):
    r = jnp.dot(x_ref[...], w_ref[...], preferred_element_type=jnp.float32)
    for cc in range(C):
        o_ref[cc] = r[:, cc * Dc:(cc + 1) * Dc]


def _tc_matmul_chunked(x, W, C):
    N, KD = x.shape
    D = W.shape[1]
    Dc = D // C
    BN = 512
    return pl.pallas_call(
        functools.partial(_mm_body, C, Dc),
        grid=(N // BN,),
        in_specs=[pl.BlockSpec((BN, KD), lambda i: (i, 0)),
                  pl.BlockSpec((KD, D), lambda i: (0, 0))],
        out_specs=pl.BlockSpec((C, BN, Dc), lambda i: (0, i, 0)),
        out_shape=jax.ShapeDtypeStruct((C, N, Dc), jnp.float32),
    )(x, W)


def _combine_mm_body(Cin, Cout, Dc, agg_ref, hc_ref, dis_ref, b_ref, w_ref,
                     o_ref):
    dinv = dis_ref[...] * dis_ref[...]
    acc = None
    for c in range(Cin):
        part = (agg_ref[c, 0] + agg_ref[c, 1] + hc_ref[c] * dinv
                + b_ref[0, pl.ds(c * Dc, Dc)][None, :])
        part = jnp.maximum(part, 0.0)
        pc = jnp.dot(part, w_ref[pl.ds(c * Dc, Dc), :],
                     preferred_element_type=jnp.float32)
        acc = pc if acc is None else acc + pc
    Dco = acc.shape[1] // Cout
    for cc in range(Cout):
        o_ref[cc] = acc[:, cc * Dco:(cc + 1) * Dco]


def _tc_combine_mm(agg, hc, dis, b, W, Cout):
    Cin, _, N, Dc = agg.shape
    D = W.shape[1]
    Dco = D // Cout
    BN = 512
    return pl.pallas_call(
        functools.partial(_combine_mm_body, Cin, Cout, Dc),
        grid=(N // BN,),
        in_specs=[pl.BlockSpec((Cin, NC, BN, Dc), lambda i: (0, 0, i, 0)),
                  pl.BlockSpec((Cin, BN, Dc), lambda i: (0, i, 0)),
                  pl.BlockSpec((BN, 1), lambda i: (i, 0)),
                  pl.BlockSpec((1, Cin * Dc), lambda i: (0, 0)),
                  pl.BlockSpec((Cin * Dc, D), lambda i: (0, 0))],
        out_specs=pl.BlockSpec((Cout, BN, Dco), lambda i: (0, i, 0)),
        out_shape=jax.ShapeDtypeStruct((Cout, N, Dco), jnp.float32),
    )(agg, hc, dis, b, W)


def _combine_body(Cin, Dc, agg_ref, hc_ref, dis_ref, b_ref, o_ref):
    dinv = dis_ref[...] * dis_ref[...]
    for c in range(Cin):
        part = (agg_ref[c, 0] + agg_ref[c, 1] + hc_ref[c] * dinv
                + b_ref[0, pl.ds(c * Dc, Dc)][None, :])
        o_ref[:, pl.ds(c * Dc, Dc)] = jnp.maximum(part, 0.0)


def _tc_combine(agg, hc, dis, b):
    Cin, _, N, Dc = agg.shape
    BN = 512
    return pl.pallas_call(
        functools.partial(_combine_body, Cin, Dc),
        grid=(N // BN,),
        in_specs=[pl.BlockSpec((Cin, NC, BN, Dc), lambda i: (0, 0, i, 0)),
                  pl.BlockSpec((Cin, BN, Dc), lambda i: (0, i, 0)),
                  pl.BlockSpec((BN, 1), lambda i: (i, 0)),
                  pl.BlockSpec((1, Cin * Dc), lambda i: (0, 0))],
        out_specs=pl.BlockSpec((BN, Cin * Dc), lambda i: (i, 0)),
        out_shape=jax.ShapeDtypeStruct((N, Cin * Dc), jnp.float32),
    )(agg, hc, dis, b)


def _head_body(l_ref, w1_ref, b1_ref, w2_ref, b2_ref, o_ref, acc_ref):
    kk = pl.program_id(0)

    @pl.when(kk == 0)
    def _():
        acc_ref[...] = jnp.zeros_like(acc_ref)

    acc_ref[...] += jnp.dot(l_ref[...], w1_ref[...],
                            preferred_element_type=jnp.float32)

    @pl.when(kk == pl.num_programs(0) - 1)
    def _():
        r = jnp.maximum(acc_ref[...] + b1_ref[...], 0.0)
        o_ref[...] = jnp.dot(r, w2_ref[...],
                             preferred_element_type=jnp.float32) + b2_ref[...]


def _tc_head(L, fc1_W, fc1_b, fc2_W, fc2_b):
    Bsz, KD = L.shape
    H = fc1_W.shape[1]
    O = fc2_W.shape[1]
    BK = 2048
    return pl.pallas_call(
        _head_body,
        grid=(KD // BK,),
        in_specs=[pl.BlockSpec((Bsz, BK), lambda k_: (0, k_)),
                  pl.BlockSpec((BK, H), lambda k_: (k_, 0)),
                  pl.BlockSpec((1, H), lambda k_: (0, 0)),
                  pl.BlockSpec((H, O), lambda k_: (0, 0)),
                  pl.BlockSpec((1, O), lambda k_: (0, 0))],
        out_specs=pl.BlockSpec((Bsz, O), lambda k_: (0, 0)),
        out_shape=jax.ShapeDtypeStruct((Bsz, O), jnp.float32),
        scratch_shapes=[pltpu.VMEM((Bsz, H), jnp.float32)],
    )(L, fc1_W, fc1_b, fc2_W, fc2_b)


def kernel(x, edge_index, batch, edge_attr, W1, b1, W2, b2,
           fc1_W, fc1_b, fc2_W, fc2_b):
    N = x.shape[0]
    E = edge_index.shape[1]
    row = edge_index[0]
    col = edge_index[1]

    deg = _sc_deg(col, edge_attr, N)
    dis = _tc_rsqrt(deg, N)
    norm = _sc_norm(dis, row, col, edge_attr, N)
    dis2 = dis.reshape(N, 1)
    col4 = col.reshape(NW, (E // NW) // K, 1, K)
    zeros = jnp.zeros((N, 64), jnp.float32)

    C1 = W1.shape[1] // 64
    h1c = _tc_matmul_chunked(x, W1, C1)
    agg1 = _sc_spmm([h1c[i] for i in range(C1)], row, col4, norm, zeros, N)

    C2 = W2.shape[1] // 64
    h2c = _tc_combine_mm(agg1, h1c, dis2, b1.reshape(1, -1), W2, C2)
    agg2 = _sc_spmm([h2c[i] for i in range(C2)], row, col4, norm, zeros, N)

    hf = _tc_combine(agg2, h2c, dis2, b2.reshape(1, -1))

    npg = fc1_W.shape[0] // hf.shape[1]
    Bsz = N // npg
    L = hf.reshape(Bsz, -1)
    return _tc_head(L, fc1_W, fc1_b.reshape(1, -1),
                    fc2_W, fc2_b.reshape(1, -1))


# speedup vs baseline: 13.9159x; 1.0329x over previous
import functools

import jax
import jax.numpy as jnp
from jax import lax
from jax.experimental import pallas as pl
from jax.experimental.pallas import tpu as pltpu
from jax.experimental.pallas import tpu_sc as plsc

NC = 2
NS = 16
NW = NC * NS
LANES = 16
K = 128


def _mesh():
    return plsc.VectorSubcoreMesh(
        core_axis_name="c", subcore_axis_name="s",
        num_cores=NC, num_subcores=NS)


_SC_PARAMS = pltpu.CompilerParams(needs_layout_passes=False,
                                 use_tc_tiling_on_sc=False)


def _sc_prep(row, col, ea, n):
    E = col.shape[0]
    Ed = E // NS
    nbd = Ed // K
    Et = E // NW
    rpt = n // NS
    share = n // NW
    nvec = n // LANES

    @functools.partial(
        pl.kernel, mesh=_mesh(), compiler_params=_SC_PARAMS,
        out_type=(jax.ShapeDtypeStruct((E,), jnp.float32),
                  jax.ShapeDtypeStruct((n,), jnp.float32)),
        scratch_types=[
            pltpu.VMEM_SHARED((n,), jnp.float32),
            pltpu.VMEM((n,), jnp.float32),
            pltpu.VMEM((nbd, 1, K), jnp.int32),
            pltpu.VMEM((Ed,), jnp.float32),
            pltpu.VMEM((Et // K, 1, K), jnp.int32),
            pltpu.VMEM((Et // K, 1, K), jnp.int32),
            pltpu.VMEM((Et,), jnp.float32),
            pltpu.VMEM((Et,), jnp.float32),
        ],
    )
    def k(col3_h, row3_h, ea_h, zeros_h, norm_h, dis_h,
          acc, p, cidx3, vals, ridx3, cidxn, eav, nbuf):
        cid = lax.axis_index("c")
        sid = lax.axis_index("s")
        w = cid * NS + sid

        pltpu.sync_copy(zeros_h.at[pl.ds(sid * rpt, rpt)],
                        acc.at[pl.ds(sid * rpt, rpt)])
        plsc.subcore_barrier()
        pltpu.sync_copy(col3_h.at[sid], cidx3)
        pltpu.sync_copy(ea_h.at[pl.ds(sid * Ed, Ed)], vals)

        def dbody(b, _):
            pltpu.sync_copy(vals.at[pl.ds(b * K, K)], acc.at[cidx3.at[b, 0]],
                            add=True)
            return 0
        lax.fori_loop(0, nbd, dbody, 0)
        plsc.subcore_barrier()

        pltpu.sync_copy(acc, p)

        @plsc.parallel_loop(0, nvec, 1, unroll=2)
        def _(i):
            sl = pl.ds(i * LANES, LANES)
            d = p[sl] + 1.0
            xh = d * 0.5
            ii = plsc.bitcast(d, jnp.int32)
            ii = jnp.int32(0x5F3759DF) - lax.shift_right_logical(ii, 1)
            y = plsc.bitcast(ii, jnp.float32)
            y = y * (1.5 - xh * y * y)
            y = y * (1.5 - xh * y * y)
            y = y * (1.5 - xh * y * y)
            p[sl] = y
        pltpu.sync_copy(p.at[pl.ds(w * share, share)],
                        dis_h.at[pl.ds(w * share, share)])

        base0 = w * Et
        nrows = Et // K
        i3 = w // NC
        o3 = lax.rem(w, NC) * nrows
        pltpu.sync_copy(row3_h.at[i3, pl.ds(o3, nrows)], ridx3)
        pltpu.sync_copy(col3_h.at[i3, pl.ds(o3, nrows)], cidxn)
        pltpu.sync_copy(ea_h.at[pl.ds(base0, Et)], eav)

        @plsc.parallel_loop(0, nrows, 1)
        def _(q):
            for r in range(K // LANES):
                sl = pl.ds(r * LANES, LANES)
                dr = plsc.load_gather(p, [ridx3[q, 0, sl]])
                dc = plsc.load_gather(p, [cidxn[q, 0, sl]])
                nbuf[pl.ds(q * K + r * LANES, LANES)] = (
                    dr * eav[pl.ds(q * K + r * LANES, LANES)] * dc)
        pltpu.sync_copy(nbuf, norm_h.at[pl.ds(base0, Et)])

    col3 = col.reshape(NS, nbd, 1, K)
    row3 = row.reshape(NS, nbd, 1, K)
    return k(col3, row3, ea, jnp.zeros((n,), jnp.float32))


def _sc_spmm(h_chunks, row, col4, norm, zeros, n):
    C = len(h_chunks)
    E = row.shape[0]
    Dc = h_chunks[0].shape[1]
    NV = Dc // LANES
    Et = E // NW
    nb = Et // K
    rpt = n // NS

    @functools.partial(
        pl.kernel, mesh=_mesh(), compiler_params=_SC_PARAMS,
        out_type=jax.ShapeDtypeStruct((C, NC, n, Dc), jnp.float32),
        scratch_types=(
            [pltpu.VMEM_SHARED((n, Dc), jnp.float32),
             pltpu.VMEM((Et,), jnp.int32),
             pltpu.VMEM((nb, 1, K), jnp.int32),
             pltpu.VMEM((Et,), jnp.float32)]
            + [pltpu.VMEM((K, Dc), jnp.float32)] * 2
            + [pltpu.SemaphoreType.DMA] * 4
        ),
    )
    def k(*refs):
        h_refs = refs[:C]
        row_h, col_h, norm_h, zeros_h, out_h = refs[C:C + 5]
        (acc, ridx, cidx, nrm, rows0, rows1,
         sem0, sem1, ssem0, ssem1) = refs[C + 5:]
        rows = (rows0, rows1)
        sems = (sem0, sem1)
        ssems = (ssem0, ssem1)
        cid = lax.axis_index("c")
        sid = lax.axis_index("s")
        w = cid * NS + sid
        estart = w * Et

        pltpu.sync_copy(row_h.at[pl.ds(estart, Et)], ridx)
        pltpu.sync_copy(col_h.at[w], cidx)
        pltpu.sync_copy(norm_h.at[pl.ds(estart, Et)], nrm)

        def scale_batch(b, buf):
            @plsc.parallel_loop(0, K // LANES, 1, unroll=2)
            def _(g):
                n16 = nrm[pl.ds(b * K + g * LANES, LANES)]
                for u in range(LANES):
                    s = n16[u]
                    e = g * LANES + u
                    for j in range(NV):
                        sl = pl.ds(j * LANES, LANES)
                        buf[e, sl] = buf[e, sl] * s

        for c in range(C):
            pltpu.sync_copy(zeros_h.at[pl.ds(sid * rpt, rpt)],
                            acc.at[pl.ds(sid * rpt, rpt)])
            plsc.subcore_barrier()
            pltpu.async_copy(h_refs[c].at[ridx.at[pl.ds(0, K)]],
                             rows0, sem0)

            def body(b, _):
                slot = lax.rem(b, 2)
                for s_ in range(2):
                    o_ = 1 - s_

                    @pl.when(slot == s_)
                    def _():
                        @pl.when(b + 1 < nb)
                        def _():
                            @pl.when(b >= 1)
                            def _():
                                pltpu.make_async_copy(
                                    rows[o_], acc.at[cidx.at[0, 0]],
                                    ssems[o_]).wait()
                            pltpu.async_copy(
                                h_refs[c].at[ridx.at[pl.ds((b + 1) * K, K)]],
                                rows[o_], sems[o_])
                        pltpu.make_async_copy(
                            h_refs[c].at[ridx.at[pl.ds(0, K)]],
                            rows[s_], sems[s_]).wait()
                        scale_batch(b, rows[s_])
                        pltpu.async_copy(rows[s_], acc.at[cidx.at[b, 0]],
                                         ssems[s_], add=True)
                return 0
            lax.fori_loop(0, nb, body, 0)
            for s_ in range(2):
                pltpu.make_async_copy(rows[s_], acc.at[cidx.at[0, 0]],
                                      ssems[s_]).wait()
            plsc.subcore_barrier()
            pltpu.sync_copy(acc.at[pl.ds(sid * rpt, rpt)],
                            out_h.at[c, cid, pl.ds(sid * rpt, rpt)])

    return k(*h_chunks, row, col4, norm, zeros)


def _mm_body(C, Dc, x_ref, w_ref, o_ref):
    r = jnp.dot(x_ref[...], w_ref[...], preferred_element_type=jnp.float32)
    for cc in range(C):
        o_ref[cc] = r[:, cc * Dc:(cc + 1) * Dc]


def _tc_matmul_chunked(x, W, C):
    N, KD = x.shape
    D = W.shape[1]
    Dc = D // C
    BN = 512
    return pl.pallas_call(
        functools.partial(_mm_body, C, Dc),
        grid=(N // BN,),
        in_specs=[pl.BlockSpec((BN, KD), lambda i: (i, 0)),
                  pl.BlockSpec((KD, D), lambda i: (0, 0))],
        out_specs=pl.BlockSpec((C, BN, Dc), lambda i: (0, i, 0)),
        out_shape=jax.ShapeDtypeStruct((C, N, Dc), jnp.float32),
    )(x, W)


def _combine_mm_body(Cin, Cout, Dc, agg_ref, hc_ref, dis_ref, b_ref, w_ref,
                     o_ref):
    dinv = dis_ref[...] * dis_ref[...]
    acc = None
    for c in range(Cin):
        part = (agg_ref[c, 0] + agg_ref[c, 1] + hc_ref[c] * dinv
                + b_ref[0, pl.ds(c * Dc, Dc)][None, :])
        part = jnp.maximum(part, 0.0)
        pc = jnp.dot(part, w_ref[pl.ds(c * Dc, Dc), :],
                     preferred_element_type=jnp.float32)
        acc = pc if acc is None else acc + pc
    Dco = acc.shape[1] // Cout
    for cc in range(Cout):
        o_ref[cc] = acc[:, cc * Dco:(cc + 1) * Dco]


def _tc_combine_mm(agg, hc, dis, b, W, Cout):
    Cin, _, N, Dc = agg.shape
    D = W.shape[1]
    Dco = D // Cout
    BN = 512
    return pl.pallas_call(
        functools.partial(_combine_mm_body, Cin, Cout, Dc),
        grid=(N // BN,),
        in_specs=[pl.BlockSpec((Cin, NC, BN, Dc), lambda i: (0, 0, i, 0)),
                  pl.BlockSpec((Cin, BN, Dc), lambda i: (0, i, 0)),
                  pl.BlockSpec((BN, 1), lambda i: (i, 0)),
                  pl.BlockSpec((1, Cin * Dc), lambda i: (0, 0)),
                  pl.BlockSpec((Cin * Dc, D), lambda i: (0, 0))],
        out_specs=pl.BlockSpec((Cout, BN, Dco), lambda i: (0, i, 0)),
        out_shape=jax.ShapeDtypeStruct((Cout, N, Dco), jnp.float32),
    )(agg, hc, dis, b, W)


def _combine_body(Cin, Dc, agg_ref, hc_ref, dis_ref, b_ref, o_ref):
    dinv = dis_ref[...] * dis_ref[...]
    for c in range(Cin):
        part = (agg_ref[c, 0] + agg_ref[c, 1] + hc_ref[c] * dinv
                + b_ref[0, pl.ds(c * Dc, Dc)][None, :])
        o_ref[:, pl.ds(c * Dc, Dc)] = jnp.maximum(part, 0.0)


def _tc_combine(agg, hc, dis, b):
    Cin, _, N, Dc = agg.shape
    BN = 512
    return pl.pallas_call(
        functools.partial(_combine_body, Cin, Dc),
        grid=(N // BN,),
        in_specs=[pl.BlockSpec((Cin, NC, BN, Dc), lambda i: (0, 0, i, 0)),
                  pl.BlockSpec((Cin, BN, Dc), lambda i: (0, i, 0)),
                  pl.BlockSpec((BN, 1), lambda i: (i, 0)),
                  pl.BlockSpec((1, Cin * Dc), lambda i: (0, 0))],
        out_specs=pl.BlockSpec((BN, Cin * Dc), lambda i: (i, 0)),
        out_shape=jax.ShapeDtypeStruct((N, Cin * Dc), jnp.float32),
    )(agg, hc, dis, b)


def _head_body(l_ref, w1_ref, b1_ref, w2_ref, b2_ref, o_ref, acc_ref):
    kk = pl.program_id(0)

    @pl.when(kk == 0)
    def _():
        acc_ref[...] = jnp.zeros_like(acc_ref)

    acc_ref[...] += jnp.dot(l_ref[...], w1_ref[...],
                            preferred_element_type=jnp.float32)

    @pl.when(kk == pl.num_programs(0) - 1)
    def _():
        r = jnp.maximum(acc_ref[...] + b1_ref[...], 0.0)
        o_ref[...] = jnp.dot(r, w2_ref[...],
                             preferred_element_type=jnp.float32) + b2_ref[...]


def _tc_head(L, fc1_W, fc1_b, fc2_W, fc2_b):
    Bsz, KD = L.shape
    H = fc1_W.shape[1]
    O = fc2_W.shape[1]
    BK = 2048
    return pl.pallas_call(
        _head_body,
        grid=(KD // BK,),
        in_specs=[pl.BlockSpec((Bsz, BK), lambda k_: (0, k_)),
                  pl.BlockSpec((BK, H), lambda k_: (k_, 0)),
                  pl.BlockSpec((1, H), lambda k_: (0, 0)),
                  pl.BlockSpec((H, O), lambda k_: (0, 0)),
                  pl.BlockSpec((1, O), lambda k_: (0, 0))],
        out_specs=pl.BlockSpec((Bsz, O), lambda k_: (0, 0)),
        out_shape=jax.ShapeDtypeStruct((Bsz, O), jnp.float32),
        scratch_shapes=[pltpu.VMEM((Bsz, H), jnp.float32)],
    )(L, fc1_W, fc1_b, fc2_W, fc2_b)


def kernel(x, edge_index, batch, edge_attr, W1, b1, W2, b2,
           fc1_W, fc1_b, fc2_W, fc2_b):
    N = x.shape[0]
    E = edge_index.shape[1]
    row = edge_index[0]
    col = edge_index[1]

    norm, dis = _sc_prep(row, col, edge_attr, N)
    dis2 = dis.reshape(N, 1)
    col4 = col.reshape(NW, (E // NW) // K, 1, K)
    zeros = jnp.zeros((N, 64), jnp.float32)

    C1 = W1.shape[1] // 64
    h1c = _tc_matmul_chunked(x, W1, C1)
    agg1 = _sc_spmm([h1c[i] for i in range(C1)], row, col4, norm, zeros, N)

    C2 = W2.shape[1] // 64
    h2c = _tc_combine_mm(agg1, h1c, dis2, b1.reshape(1, -1), W2, C2)
    agg2 = _sc_spmm([h2c[i] for i in range(C2)], row, col4, norm, zeros, N)

    hf = _tc_combine(agg2, h2c, dis2, b2.reshape(1, -1))

    npg = fc1_W.shape[0] // hf.shape[1]
    Bsz = N // npg
    L = hf.reshape(Bsz, -1)
    return _tc_head(L, fc1_W, fc1_b.reshape(1, -1),
                    fc2_W, fc2_b.reshape(1, -1))


# speedup vs baseline: 14.7328x; 1.0587x over previous
import functools

import jax
import jax.numpy as jnp
from jax import lax
from jax.experimental import pallas as pl
from jax.experimental.pallas import tpu as pltpu
from jax.experimental.pallas import tpu_sc as plsc

NC = 2
NS = 16
NW = NC * NS
LANES = 16
K = 128


def _mesh():
    return plsc.VectorSubcoreMesh(
        core_axis_name="c", subcore_axis_name="s",
        num_cores=NC, num_subcores=NS)


_SC_PARAMS = pltpu.CompilerParams(needs_layout_passes=False,
                                 use_tc_tiling_on_sc=False)


def _sc_prep(row, col, ea, n):
    E = col.shape[0]
    Ed = E // NS
    nbd = Ed // K
    Et = E // NW
    rpt = n // NS
    share = n // NW
    nvec = n // LANES

    @functools.partial(
        pl.kernel, mesh=_mesh(), compiler_params=_SC_PARAMS,
        out_type=(jax.ShapeDtypeStruct((E,), jnp.float32),
                  jax.ShapeDtypeStruct((n,), jnp.float32)),
        scratch_types=[
            pltpu.VMEM_SHARED((n,), jnp.float32),
            pltpu.VMEM((n,), jnp.float32),
            pltpu.VMEM((nbd, 1, K), jnp.int32),
            pltpu.VMEM((Ed,), jnp.float32),
            pltpu.VMEM((Et // K, 1, K), jnp.int32),
            pltpu.VMEM((Et // K, 1, K), jnp.int32),
            pltpu.VMEM((Et,), jnp.float32),
            pltpu.VMEM((Et,), jnp.float32),
        ],
    )
    def k(col3_h, row3_h, ea_h, zeros_h, norm_h, dis_h,
          acc, p, cidx3, vals, ridx3, cidxn, eav, nbuf):
        cid = lax.axis_index("c")
        sid = lax.axis_index("s")
        w = cid * NS + sid

        pltpu.sync_copy(zeros_h.at[pl.ds(sid * rpt, rpt)],
                        acc.at[pl.ds(sid * rpt, rpt)])
        plsc.subcore_barrier()
        pltpu.sync_copy(col3_h.at[sid], cidx3)
        pltpu.sync_copy(ea_h.at[pl.ds(sid * Ed, Ed)], vals)

        def dbody(b, _):
            pltpu.sync_copy(vals.at[pl.ds(b * K, K)], acc.at[cidx3.at[b, 0]],
                            add=True)
            return 0
        lax.fori_loop(0, nbd, dbody, 0)
        plsc.subcore_barrier()

        pltpu.sync_copy(acc, p)

        @plsc.parallel_loop(0, nvec, 1, unroll=2)
        def _(i):
            sl = pl.ds(i * LANES, LANES)
            d = p[sl] + 1.0
            xh = d * 0.5
            ii = plsc.bitcast(d, jnp.int32)
            ii = jnp.int32(0x5F3759DF) - lax.shift_right_logical(ii, 1)
            y = plsc.bitcast(ii, jnp.float32)
            y = y * (1.5 - xh * y * y)
            y = y * (1.5 - xh * y * y)
            y = y * (1.5 - xh * y * y)
            p[sl] = y
        pltpu.sync_copy(p.at[pl.ds(w * share, share)],
                        dis_h.at[pl.ds(w * share, share)])

        base0 = w * Et
        nrows = Et // K
        i3 = w // NC
        o3 = lax.rem(w, NC) * nrows
        pltpu.sync_copy(row3_h.at[i3, pl.ds(o3, nrows)], ridx3)
        pltpu.sync_copy(col3_h.at[i3, pl.ds(o3, nrows)], cidxn)
        pltpu.sync_copy(ea_h.at[pl.ds(base0, Et)], eav)

        @plsc.parallel_loop(0, nrows, 1)
        def _(q):
            for r in range(K // LANES):
                sl = pl.ds(r * LANES, LANES)
                dr = plsc.load_gather(p, [ridx3[q, 0, sl]])
                dc = plsc.load_gather(p, [cidxn[q, 0, sl]])
                nbuf[pl.ds(q * K + r * LANES, LANES)] = (
                    dr * eav[pl.ds(q * K + r * LANES, LANES)] * dc)
        pltpu.sync_copy(nbuf, norm_h.at[pl.ds(base0, Et)])

    col3 = col.reshape(NS, nbd, 1, K)
    row3 = row.reshape(NS, nbd, 1, K)
    return k(col3, row3, ea, jnp.zeros((n,), jnp.float32))


def _sc_spmm(h_chunks, row, col4, norm, zeros, n):
    C = len(h_chunks)
    E = row.shape[0]
    Dc = h_chunks[0].shape[1]
    NV = Dc // LANES
    Et = E // NW
    nb = Et // K
    rpt = n // NS

    KG = 256
    SB = KG // K
    nbg = Et // KG

    @functools.partial(
        pl.kernel, mesh=_mesh(), compiler_params=_SC_PARAMS,
        out_type=jax.ShapeDtypeStruct((C, NC, n, Dc), jnp.float32),
        scratch_types=(
            [pltpu.VMEM_SHARED((n, Dc), jnp.float32),
             pltpu.VMEM((Et,), jnp.int32),
             pltpu.VMEM((nb, 1, K), jnp.int32),
             pltpu.VMEM((Et,), jnp.float32)]
            + [pltpu.VMEM((KG, Dc), jnp.float32)] * 2
            + [pltpu.SemaphoreType.DMA] * 4
        ),
    )
    def k(*refs):
        h_refs = refs[:C]
        row_h, col_h, norm_h, zeros_h, out_h = refs[C:C + 5]
        (acc, ridx, cidx, nrm, rows0, rows1,
         sem0, sem1, ssem0, ssem1) = refs[C + 5:]
        rows = (rows0, rows1)
        sems = (sem0, sem1)
        ssems = (ssem0, ssem1)
        cid = lax.axis_index("c")
        sid = lax.axis_index("s")
        w = cid * NS + sid
        estart = w * Et

        pltpu.sync_copy(row_h.at[pl.ds(estart, Et)], ridx)
        pltpu.sync_copy(col_h.at[w], cidx)
        pltpu.sync_copy(norm_h.at[pl.ds(estart, Et)], nrm)

        def scale_batch(b, buf):
            @plsc.parallel_loop(0, KG // LANES, 1, unroll=2)
            def _(g):
                n16 = nrm[pl.ds(b * KG + g * LANES, LANES)]
                for u in range(LANES):
                    s = n16[u]
                    e = g * LANES + u
                    for j in range(NV):
                        sl = pl.ds(j * LANES, LANES)
                        buf[e, sl] = buf[e, sl] * s

        def drain_scatters(s_):
            for i in range(SB):
                pltpu.make_async_copy(rows[s_].at[pl.ds(i * K, K)],
                                      acc.at[cidx.at[0, 0]],
                                      ssems[s_]).wait()

        for c in range(C):
            pltpu.sync_copy(zeros_h.at[pl.ds(sid * rpt, rpt)],
                            acc.at[pl.ds(sid * rpt, rpt)])
            plsc.subcore_barrier()
            pltpu.async_copy(h_refs[c].at[ridx.at[pl.ds(0, KG)]],
                             rows0, sem0)

            def body(b, _):
                slot = lax.rem(b, 2)
                for s_ in range(2):
                    o_ = 1 - s_

                    @pl.when(slot == s_)
                    def _():
                        @pl.when(b + 1 < nbg)
                        def _():
                            @pl.when(b >= 1)
                            def _():
                                drain_scatters(o_)
                            pltpu.async_copy(
                                h_refs[c].at[ridx.at[pl.ds((b + 1) * KG, KG)]],
                                rows[o_], sems[o_])
                        pltpu.make_async_copy(
                            h_refs[c].at[ridx.at[pl.ds(0, KG)]],
                            rows[s_], sems[s_]).wait()
                        scale_batch(b, rows[s_])
                        for i in range(SB):
                            pltpu.async_copy(
                                rows[s_].at[pl.ds(i * K, K)],
                                acc.at[cidx.at[b * SB + i, 0]],
                                ssems[s_], add=True)
                return 0
            lax.fori_loop(0, nbg, body, 0)
            for s_ in range(2):
                drain_scatters(s_)
            plsc.subcore_barrier()
            pltpu.sync_copy(acc.at[pl.ds(sid * rpt, rpt)],
                            out_h.at[c, cid, pl.ds(sid * rpt, rpt)])

    return k(*h_chunks, row, col4, norm, zeros)


def _mm_body(C, Dc, x_ref, w_ref, o_ref):
    r = jnp.dot(x_ref[...], w_ref[...], preferred_element_type=jnp.float32)
    for cc in range(C):
        o_ref[cc] = r[:, cc * Dc:(cc + 1) * Dc]


def _tc_matmul_chunked(x, W, C):
    N, KD = x.shape
    D = W.shape[1]
    Dc = D // C
    BN = 512
    return pl.pallas_call(
        functools.partial(_mm_body, C, Dc),
        grid=(N // BN,),
        in_specs=[pl.BlockSpec((BN, KD), lambda i: (i, 0)),
                  pl.BlockSpec((KD, D), lambda i: (0, 0))],
        out_specs=pl.BlockSpec((C, BN, Dc), lambda i: (0, i, 0)),
        out_shape=jax.ShapeDtypeStruct((C, N, Dc), jnp.float32),
    )(x, W)


def _combine_mm_body(Cin, Cout, Dc, agg_ref, hc_ref, dis_ref, b_ref, w_ref,
                     o_ref):
    dinv = dis_ref[...] * dis_ref[...]
    acc = None
    for c in range(Cin):
        part = (agg_ref[c, 0] + agg_ref[c, 1] + hc_ref[c] * dinv
                + b_ref[0, pl.ds(c * Dc, Dc)][None, :])
        part = jnp.maximum(part, 0.0)
        pc = jnp.dot(part, w_ref[pl.ds(c * Dc, Dc), :],
                     preferred_element_type=jnp.float32)
        acc = pc if acc is None else acc + pc
    Dco = acc.shape[1] // Cout
    for cc in range(Cout):
        o_ref[cc] = acc[:, cc * Dco:(cc + 1) * Dco]


def _tc_combine_mm(agg, hc, dis, b, W, Cout):
    Cin, _, N, Dc = agg.shape
    D = W.shape[1]
    Dco = D // Cout
    BN = 512
    return pl.pallas_call(
        functools.partial(_combine_mm_body, Cin, Cout, Dc),
        grid=(N // BN,),
        in_specs=[pl.BlockSpec((Cin, NC, BN, Dc), lambda i: (0, 0, i, 0)),
                  pl.BlockSpec((Cin, BN, Dc), lambda i: (0, i, 0)),
                  pl.BlockSpec((BN, 1), lambda i: (i, 0)),
                  pl.BlockSpec((1, Cin * Dc), lambda i: (0, 0)),
                  pl.BlockSpec((Cin * Dc, D), lambda i: (0, 0))],
        out_specs=pl.BlockSpec((Cout, BN, Dco), lambda i: (0, i, 0)),
        out_shape=jax.ShapeDtypeStruct((Cout, N, Dco), jnp.float32),
    )(agg, hc, dis, b, W)


def _combine_body(Cin, Dc, agg_ref, hc_ref, dis_ref, b_ref, o_ref):
    dinv = dis_ref[...] * dis_ref[...]
    for c in range(Cin):
        part = (agg_ref[c, 0] + agg_ref[c, 1] + hc_ref[c] * dinv
                + b_ref[0, pl.ds(c * Dc, Dc)][None, :])
        o_ref[:, pl.ds(c * Dc, Dc)] = jnp.maximum(part, 0.0)


def _tc_combine(agg, hc, dis, b):
    Cin, _, N, Dc = agg.shape
    BN = 512
    return pl.pallas_call(
        functools.partial(_combine_body, Cin, Dc),
        grid=(N // BN,),
        in_specs=[pl.BlockSpec((Cin, NC, BN, Dc), lambda i: (0, 0, i, 0)),
                  pl.BlockSpec((Cin, BN, Dc), lambda i: (0, i, 0)),
                  pl.BlockSpec((BN, 1), lambda i: (i, 0)),
                  pl.BlockSpec((1, Cin * Dc), lambda i: (0, 0))],
        out_specs=pl.BlockSpec((BN, Cin * Dc), lambda i: (i, 0)),
        out_shape=jax.ShapeDtypeStruct((N, Cin * Dc), jnp.float32),
    )(agg, hc, dis, b)


def _head_body(l_ref, w1_ref, b1_ref, w2_ref, b2_ref, o_ref, acc_ref):
    kk = pl.program_id(0)

    @pl.when(kk == 0)
    def _():
        acc_ref[...] = jnp.zeros_like(acc_ref)

    acc_ref[...] += jnp.dot(l_ref[...], w1_ref[...],
                            preferred_element_type=jnp.float32)

    @pl.when(kk == pl.num_programs(0) - 1)
    def _():
        r = jnp.maximum(acc_ref[...] + b1_ref[...], 0.0)
        o_ref[...] = jnp.dot(r, w2_ref[...],
                             preferred_element_type=jnp.float32) + b2_ref[...]


def _tc_head(L, fc1_W, fc1_b, fc2_W, fc2_b):
    Bsz, KD = L.shape
    H = fc1_W.shape[1]
    O = fc2_W.shape[1]
    BK = 2048
    return pl.pallas_call(
        _head_body,
        grid=(KD // BK,),
        in_specs=[pl.BlockSpec((Bsz, BK), lambda k_: (0, k_)),
                  pl.BlockSpec((BK, H), lambda k_: (k_, 0)),
                  pl.BlockSpec((1, H), lambda k_: (0, 0)),
                  pl.BlockSpec((H, O), lambda k_: (0, 0)),
                  pl.BlockSpec((1, O), lambda k_: (0, 0))],
        out_specs=pl.BlockSpec((Bsz, O), lambda k_: (0, 0)),
        out_shape=jax.ShapeDtypeStruct((Bsz, O), jnp.float32),
        scratch_shapes=[pltpu.VMEM((Bsz, H), jnp.float32)],
    )(L, fc1_W, fc1_b, fc2_W, fc2_b)


def kernel(x, edge_index, batch, edge_attr, W1, b1, W2, b2,
           fc1_W, fc1_b, fc2_W, fc2_b):
    N = x.shape[0]
    E = edge_index.shape[1]
    row = edge_index[0]
    col = edge_index[1]

    norm, dis = _sc_prep(row, col, edge_attr, N)
    dis2 = dis.reshape(N, 1)
    col4 = col.reshape(NW, (E // NW) // K, 1, K)
    zeros = jnp.zeros((N, 64), jnp.float32)

    C1 = W1.shape[1] // 64
    h1c = _tc_matmul_chunked(x, W1, C1)
    agg1 = _sc_spmm([h1c[i] for i in range(C1)], row, col4, norm, zeros, N)

    C2 = W2.shape[1] // 64
    h2c = _tc_combine_mm(agg1, h1c, dis2, b1.reshape(1, -1), W2, C2)
    agg2 = _sc_spmm([h2c[i] for i in range(C2)], row, col4, norm, zeros, N)

    hf = _tc_combine(agg2, h2c, dis2, b2.reshape(1, -1))

    npg = fc1_W.shape[0] // hf.shape[1]
    Bsz = N // npg
    L = hf.reshape(Bsz, -1)
    return _tc_head(L, fc1_W, fc1_b.reshape(1, -1),
                    fc2_W, fc2_b.reshape(1, -1))
